# baseline (speedup 1.0000x reference)
import functools

import jax
import jax.numpy as jnp
from jax import lax
from jax.experimental import pallas as pl
from jax.experimental.pallas import tpu as pltpu
from jax.experimental.pallas import tpu_sc as plsc

NUM_NODES = 100000
FEAT_DIM = 128
EMB_DIM = 64
BATCH = 16384
OUT_DIM = FEAT_DIM + EMB_DIM

NC = 2
NS = 16
NW = NC * NS
BPW = BATCH // NW
NCHUNK = 4
C = BPW // NCHUNK
L = 16
G = C // L

_mesh = plsc.VectorSubcoreMesh(core_axis_name="c", subcore_axis_name="s")


@functools.partial(
    pl.kernel,
    mesh=_mesh,
    out_type=jax.ShapeDtypeStruct((OUT_DIM, BATCH), jnp.float32),
    scratch_types=[
        pltpu.VMEM((BPW,), jnp.int32),
        pltpu.VMEM((BPW,), jnp.int32),
        pltpu.VMEM((2, C, FEAT_DIM), jnp.float32),
        pltpu.VMEM((2, C, 2 * EMB_DIM), jnp.float32),
        pltpu.VMEM((2, FEAT_DIM, C), jnp.float32),
        pltpu.VMEM((2, EMB_DIM, C), jnp.float32),
        pltpu.SemaphoreType.DMA,
        pltpu.SemaphoreType.DMA,
        pltpu.SemaphoreType.DMA,
    ],
    compiler_params=pltpu.CompilerParams(needs_layout_passes=False),
)
def _encoder(idx_hbm, feat_hbm, emb2_hbm, out_hbm, idx_v, ix2_v, fbuf, ebuf,
             tf, te, gsem0, gsem1, wsem):
    wid = lax.axis_index("s") * NC + lax.axis_index("c")
    base = wid * BPW
    pltpu.sync_copy(idx_hbm.at[pl.ds(base, BPW)], idx_v)

    def pair_ix(i, carry):
        v = idx_v[pl.ds(i * L, L)]
        ix2_v[pl.ds(i * L, L)] = lax.shift_right_logical(v, 1)
        return carry

    lax.fori_loop(0, BPW // L, pair_ix, 0, unroll=4)

    gsems = (gsem0, gsem1)

    def start_gathers(j):
        s = j % 2
        cf = pltpu.async_copy(
            feat_hbm.at[idx_v.at[pl.ds(j * C, C)]], fbuf.at[s], gsems[s])
        ce = pltpu.async_copy(
            emb2_hbm.at[ix2_v.at[pl.ds(j * C, C)]], ebuf.at[s], gsems[s])
        return cf, ce

    pending = start_gathers(0)
    writes = []
    for j in range(NCHUNK):
        s = j % 2
        cf, ce = pending
        cf.wait()
        ce.wait()
        if j + 1 < NCHUNK:
            pending = start_gathers(j + 1)

        for g in range(G):
            rvec = lax.iota(jnp.int32, L) + (g * L)

            def tr_feat(f, carry, rvec=rvec, g=g, s=s):
                fvec = jnp.full((L,), f, jnp.int32)
                v = plsc.load_gather(fbuf.at[s], [rvec, fvec])
                tf[s, f, pl.ds(g * L, L)] = v
                return carry

            lax.fori_loop(0, FEAT_DIM, tr_feat, 0, unroll=8)

        for g in range(G):
            rvec = lax.iota(jnp.int32, L) + (g * L)
            ivec = idx_v[pl.ds(j * C + g * L, L)]
            bvec = (ivec & 1) * EMB_DIM

            def tr_emb(f, carry, rvec=rvec, bvec=bvec, g=g, s=s):
                cvec = bvec + f
                v = plsc.load_gather(ebuf.at[s], [rvec, cvec])
                te[s, f, pl.ds(g * L, L)] = v
                return carry

            lax.fori_loop(0, EMB_DIM, tr_emb, 0, unroll=8)

        if j >= 2:
            for w in writes[j - 2]:
                w.wait()
        col = base + j * C
        wf = pltpu.async_copy(
            tf.at[s], out_hbm.at[pl.ds(0, FEAT_DIM), pl.ds(col, C)], wsem)
        we = pltpu.async_copy(
            te.at[s], out_hbm.at[pl.ds(FEAT_DIM, EMB_DIM), pl.ds(col, C)],
            wsem)
        writes.append((wf, we))

    for pair in writes[-2:]:
        for w in pair:
            w.wait()


def kernel(indices, features, emb_table):
    idx = indices.astype(jnp.int32)
    emb2 = emb_table.reshape(NUM_NODES // 2, 2 * EMB_DIM)
    out_t = _encoder(idx, features, emb2)
    return out_t.T


# speedup vs baseline: 1.4216x; 1.4216x over previous
import functools

import jax
import jax.numpy as jnp
from jax import lax
from jax.experimental import pallas as pl
from jax.experimental.pallas import tpu as pltpu
from jax.experimental.pallas import tpu_sc as plsc

NUM_NODES = 100000
FEAT_DIM = 128
EMB_DIM = 64
BATCH = 16384
OUT_DIM = FEAT_DIM + EMB_DIM

NC = 2
NS = 16
NW = NC * NS
BPW = BATCH // NW
NCHUNK = 4
C = BPW // NCHUNK
L = 16
G = C // L

_mesh = plsc.VectorSubcoreMesh(core_axis_name="c", subcore_axis_name="s")


@functools.partial(
    pl.kernel,
    mesh=_mesh,
    out_type=jax.ShapeDtypeStruct((OUT_DIM, BATCH), jnp.float32),
    scratch_types=[
        pltpu.VMEM((BPW,), jnp.int32),
        pltpu.VMEM((BPW,), jnp.int32),
        pltpu.VMEM((2, C, FEAT_DIM), jnp.float32),
        pltpu.VMEM((2, C, 2 * EMB_DIM), jnp.float32),
        pltpu.VMEM((2, FEAT_DIM, C), jnp.float32),
        pltpu.VMEM((2, EMB_DIM, C), jnp.float32),
        pltpu.SemaphoreType.DMA,
        pltpu.SemaphoreType.DMA,
        pltpu.SemaphoreType.DMA,
    ],
    compiler_params=pltpu.CompilerParams(needs_layout_passes=False),
)
def _encoder(idx_hbm, feat_hbm, emb2_hbm, out_hbm, idx_v, ix2_v, fbuf, ebuf,
             tf, te, gsem0, gsem1, wsem):
    wid = lax.axis_index("s") * NC + lax.axis_index("c")
    base = wid * BPW
    pltpu.sync_copy(idx_hbm.at[pl.ds(base, BPW)], idx_v)

    def pair_ix(i, carry):
        v = idx_v[pl.ds(i * L, L)]
        ix2_v[pl.ds(i * L, L)] = lax.shift_right_logical(v, 1)
        return carry

    lax.fori_loop(0, BPW // L, pair_ix, 0, unroll=4)

    gsems = (gsem0, gsem1)

    def start_gathers(j):
        s = j % 2
        cf = pltpu.async_copy(
            feat_hbm.at[idx_v.at[pl.ds(j * C, C)]], fbuf.at[s], gsems[s])
        ce = pltpu.async_copy(
            emb2_hbm.at[ix2_v.at[pl.ds(j * C, C)]], ebuf.at[s], gsems[s])
        return cf, ce

    pending = start_gathers(0)
    writes = []
    for j in range(NCHUNK):
        s = j % 2
        cf, ce = pending
        cf.wait()
        ce.wait()
        if j + 1 < NCHUNK:
            pending = start_gathers(j + 1)

        lane = lax.iota(jnp.int32, L)
        for g in range(G):
            rvec = lane + (g * L)

            def tr_feat(f, carry, rvec=rvec, s=s):
                cvec = (lane + f) & (FEAT_DIM - 1)
                v = plsc.load_gather(fbuf.at[s], [rvec, cvec])
                plsc.store_scatter(tf.at[s], [cvec, rvec], v)
                return carry

            lax.fori_loop(0, FEAT_DIM, tr_feat, 0, unroll=8)

        for g in range(G):
            rvec = lane + (g * L)
            ivec = idx_v[pl.ds(j * C + g * L, L)]
            bvec = (ivec & 1) * EMB_DIM

            def tr_emb(f, carry, rvec=rvec, bvec=bvec, s=s):
                cvec = (lane + f) & (EMB_DIM - 1)
                v = plsc.load_gather(ebuf.at[s], [rvec, bvec + cvec])
                plsc.store_scatter(te.at[s], [cvec, rvec], v)
                return carry

            lax.fori_loop(0, EMB_DIM, tr_emb, 0, unroll=8)

        if j >= 2:
            for w in writes[j - 2]:
                w.wait()
        col = base + j * C
        wf = pltpu.async_copy(
            tf.at[s], out_hbm.at[pl.ds(0, FEAT_DIM), pl.ds(col, C)], wsem)
        we = pltpu.async_copy(
            te.at[s], out_hbm.at[pl.ds(FEAT_DIM, EMB_DIM), pl.ds(col, C)],
            wsem)
        writes.append((wf, we))

    for pair in writes[-2:]:
        for w in pair:
            w.wait()


def kernel(indices, features, emb_table):
    idx = indices.astype(jnp.int32)
    emb2 = emb_table.reshape(NUM_NODES // 2, 2 * EMB_DIM)
    out_t = _encoder(idx, features, emb2)
    return out_t.T


# speedup vs baseline: 1.5407x; 1.0838x over previous
import functools

import jax
import jax.numpy as jnp
from jax import lax
from jax.experimental import pallas as pl
from jax.experimental.pallas import tpu as pltpu
from jax.experimental.pallas import tpu_sc as plsc

NUM_NODES = 100000
FEAT_DIM = 128
EMB_DIM = 64
BATCH = 16384
OUT_DIM = FEAT_DIM + EMB_DIM

NC = 2
NS = 16
NW = NC * NS
BPW = BATCH // NW
NCHUNK = 4
C = BPW // NCHUNK
L = 16
G = C // L

_mesh = plsc.VectorSubcoreMesh(core_axis_name="c", subcore_axis_name="s")


@functools.partial(
    pl.kernel,
    mesh=_mesh,
    out_type=jax.ShapeDtypeStruct((OUT_DIM, BATCH), jnp.float32),
    scratch_types=[
        pltpu.VMEM((BPW,), jnp.int32),
        pltpu.VMEM((2, C, FEAT_DIM), jnp.float32),
        pltpu.VMEM((2, C, FEAT_DIM), jnp.float32),
        pltpu.VMEM((2, FEAT_DIM, C), jnp.float32),
        pltpu.VMEM((2, EMB_DIM, C), jnp.float32),
        pltpu.SemaphoreType.DMA,
        pltpu.SemaphoreType.DMA,
        pltpu.SemaphoreType.DMA,
    ],
    compiler_params=pltpu.CompilerParams(needs_layout_passes=False),
)
def _encoder(idx_hbm, feat_hbm, embp_hbm, out_hbm, idx_v, fbuf, ebuf,
             tf, te, gsem0, gsem1, wsem):
    wid = lax.axis_index("s") * NC + lax.axis_index("c")
    base = wid * BPW
    pltpu.sync_copy(idx_hbm.at[pl.ds(base, BPW)], idx_v)

    gsems = (gsem0, gsem1)

    def start_gathers(j):
        s = j % 2
        ix = idx_v.at[pl.ds(j * C, C)]
        cf = pltpu.async_copy(feat_hbm.at[ix], fbuf.at[s], gsems[s])
        ce = pltpu.async_copy(embp_hbm.at[ix], ebuf.at[s], gsems[s])
        return cf, ce

    pending = start_gathers(0)
    writes = []
    lane = lax.iota(jnp.int32, L)
    for j in range(NCHUNK):
        s = j % 2
        cf, ce = pending
        cf.wait()
        ce.wait()
        if j + 1 < NCHUNK:
            pending = start_gathers(j + 1)

        for g in range(G):
            rvec = lane + (g * L)

            def tr_feat(f, carry, rvec=rvec, s=s):
                cvec = (lane + f) & (FEAT_DIM - 1)
                v = plsc.load_gather(fbuf.at[s], [rvec, cvec])
                plsc.store_scatter(tf.at[s], [cvec, rvec], v)
                return carry

            lax.fori_loop(0, FEAT_DIM, tr_feat, 0, unroll=8)

        for g in range(G):
            rvec = lane + (g * L)

            def tr_emb(f, carry, rvec=rvec, s=s):
                cvec = (lane + f) & (EMB_DIM - 1)
                v = plsc.load_gather(ebuf.at[s], [rvec, cvec])
                plsc.store_scatter(te.at[s], [cvec, rvec], v)
                return carry

            lax.fori_loop(0, EMB_DIM, tr_emb, 0, unroll=8)

        if j >= 2:
            for w in writes[j - 2]:
                w.wait()
        col = base + j * C
        wf = pltpu.async_copy(
            tf.at[s], out_hbm.at[pl.ds(0, FEAT_DIM), pl.ds(col, C)], wsem)
        we = pltpu.async_copy(
            te.at[s], out_hbm.at[pl.ds(FEAT_DIM, EMB_DIM), pl.ds(col, C)],
            wsem)
        writes.append((wf, we))

    for pair in writes[-2:]:
        for w in pair:
            w.wait()


def kernel(indices, features, emb_table):
    idx = indices.astype(jnp.int32)
    emb_p = jnp.pad(emb_table, ((0, 0), (0, FEAT_DIM - EMB_DIM)))
    out_t = _encoder(idx, features, emb_p)
    return out_t.T
